# Initial kernel scaffold; baseline (speedup 1.0000x reference)
import functools

import jax
import jax.numpy as jnp
from jax import lax
from jax.experimental import pallas as pl
from jax.experimental.pallas import tpu as pltpu
from jax.experimental.pallas import tpu_sc as plsc

N_NODES = 10000
D = 256
DH = 128
E = 160000
K = 125
ROWS = E // K
NS = 16
TROWS = ROWS // NS
NPT = N_NODES // NS
ZR = 125
CW = 16


def _sc_segment_sum(x0, x1, src2, dst2):
    mesh = plsc.VectorSubcoreMesh(core_axis_name="c", subcore_axis_name="s")

    @functools.partial(
        pl.kernel,
        mesh=mesh,
        out_type=(
            jax.ShapeDtypeStruct((N_NODES, DH), jnp.float32),
            jax.ShapeDtypeStruct((N_NODES, DH), jnp.float32),
            jax.ShapeDtypeStruct((N_NODES, CW), jnp.float32),
        ),
        scratch_types=[
            pltpu.VMEM((TROWS, K), jnp.int32),
            pltpu.VMEM((TROWS, K), jnp.int32),
            pltpu.VMEM((K, DH), jnp.float32),
            pltpu.VMEM((K, CW), jnp.float32),
            pltpu.VMEM((ZR, DH), jnp.float32),
            pltpu.VMEM((NPT, CW), jnp.float32),
            pltpu.VMEM_SHARED((N_NODES, DH), jnp.float32),
            pltpu.VMEM_SHARED((N_NODES, CW), jnp.float32),
            pltpu.SemaphoreType.DMA,
        ],
    )
    def k(x0_hbm, x1_hbm, src_hbm, dst_hbm, out0, out1, cnt_out,
          sidx, didx, rows, ones_v, zstage, cstage, acc, cnt, sem):
        c = lax.axis_index("c")
        s = lax.axis_index("s")
        base = s * NPT

        zero16 = jnp.zeros((16,), jnp.float32)
        one16 = jnp.ones((16,), jnp.float32)

        def zrow(i, carry):
            for j in range(DH // 16):
                zstage[i, pl.ds(j * 16, 16)] = zero16
            return carry

        lax.fori_loop(0, ZR, zrow, 0)

        def crow(i, carry):
            cstage[i, :] = zero16
            return carry

        lax.fori_loop(0, NPT, crow, 0)

        def orow(i, carry):
            ones_v[i, :] = one16
            return carry

        lax.fori_loop(0, K, orow, 0)

        for j in range(NPT // ZR):
            pltpu.sync_copy(zstage, acc.at[pl.ds(base + j * ZR, ZR)])
        pltpu.sync_copy(cstage, cnt.at[pl.ds(base, NPT)])
        plsc.subcore_barrier()

        pltpu.sync_copy(src_hbm.at[pl.ds(s * TROWS, TROWS)], sidx)
        pltpu.sync_copy(dst_hbm.at[pl.ds(s * TROWS, TROWS)], didx)

        def edge_loop(x_ref, with_counts):
            def body(i, carry):
                pltpu.async_copy(x_ref.at[sidx.at[i]], rows, sem).wait()
                pltpu.sync_copy(rows, acc.at[didx.at[i]], add=True)
                if with_counts:
                    pltpu.sync_copy(ones_v, cnt.at[didx.at[i]], add=True)
                return carry

            lax.fori_loop(0, TROWS, body, 0)

        @pl.when(c == 0)
        def _():
            edge_loop(x0_hbm, True)

        @pl.when(c == 1)
        def _():
            edge_loop(x1_hbm, False)

        plsc.subcore_barrier()

        def copy_out(dst_hbm_ref):
            for j in range(NPT // ZR):
                pltpu.sync_copy(acc.at[pl.ds(base + j * ZR, ZR)], zstage)
                pltpu.sync_copy(zstage, dst_hbm_ref.at[pl.ds(base + j * ZR, ZR)])

        @pl.when(c == 0)
        def _():
            copy_out(out0)
            pltpu.sync_copy(cnt.at[pl.ds(base, NPT)], cstage)
            pltpu.sync_copy(cstage, cnt_out.at[pl.ds(base, NPT)])

        @pl.when(c == 1)
        def _():
            copy_out(out1)

    return k(x0, x1, src2, dst2)


BM = 400


def _tc_body(x_ref, nb0_ref, nb1_ref, cnt_ref, wst_ref, wnt0_ref, wnt1_ref,
             b_ref, g_ref, be_ref, o_ref):
    hs = jnp.dot(x_ref[...], wst_ref[...], preferred_element_type=jnp.float32)
    hn = (jnp.dot(nb0_ref[...], wnt0_ref[...], preferred_element_type=jnp.float32)
          + jnp.dot(nb1_ref[...], wnt1_ref[...], preferred_element_type=jnp.float32))
    inv = 1.0 / jnp.maximum(cnt_ref[...], 1.0)
    h = hs + hn * inv + b_ref[...]
    mu = jnp.mean(h, axis=-1, keepdims=True)
    d = h - mu
    var = jnp.mean(d * d, axis=-1, keepdims=True)
    o_ref[...] = d * lax.rsqrt(var + 1e-5) * g_ref[...] + be_ref[...]


def _tc_combine(x, nb0, nb1, cnt, wst, wnt0, wnt1, bias, gamma, beta):
    grid = (N_NODES // BM,)
    return pl.pallas_call(
        _tc_body,
        grid=grid,
        in_specs=[
            pl.BlockSpec((BM, D), lambda i: (i, 0)),
            pl.BlockSpec((BM, DH), lambda i: (i, 0)),
            pl.BlockSpec((BM, DH), lambda i: (i, 0)),
            pl.BlockSpec((BM, 1), lambda i: (i, 0)),
            pl.BlockSpec((D, D), lambda i: (0, 0)),
            pl.BlockSpec((DH, D), lambda i: (0, 0)),
            pl.BlockSpec((DH, D), lambda i: (0, 0)),
            pl.BlockSpec((1, D), lambda i: (0, 0)),
            pl.BlockSpec((1, D), lambda i: (0, 0)),
            pl.BlockSpec((1, D), lambda i: (0, 0)),
        ],
        out_specs=pl.BlockSpec((BM, D), lambda i: (i, 0)),
        out_shape=jax.ShapeDtypeStruct((N_NODES, D), jnp.float32),
    )(x, nb0, nb1, cnt, wst, wnt0, wnt1, bias, gamma, beta)


@jax.jit
def kernel(x, edge_index, deg, W_self, W_neigh, bias, ln_gamma, ln_beta):
    del deg
    x0 = x[:, :DH]
    x1 = x[:, DH:]
    src2 = edge_index[1].reshape(ROWS, K)
    dst2 = edge_index[0].reshape(ROWS, K)
    nb0, nb1, cnt_tab = _sc_segment_sum(x0, x1, src2, dst2)
    cnt = cnt_tab[:, :1]
    wst = W_self.T
    wnt = W_neigh.T
    return _tc_combine(x, nb0, nb1, cnt, wst, wnt[:DH], wnt[DH:],
                       bias[None, :], ln_gamma[None, :], ln_beta[None, :])


# speedup vs baseline: 5.5685x; 5.5685x over previous
import functools

import jax
import jax.numpy as jnp
from jax import lax
from jax.experimental import pallas as pl
from jax.experimental.pallas import tpu as pltpu
from jax.experimental.pallas import tpu_sc as plsc

N_NODES = 10000
NPAD = 10240
D = 256
DH = 128
E = 160000
K = 125
ROWS = E // K
NS = 16
TROWS = ROWS // NS
EPT = E // NS
NPT = N_NODES // NS
G = 8
CG = 2000
CROWS = NPAD // DH


def _sc_segment_sum(x0, x1, src2, dst2, dst1):
    mesh = plsc.VectorSubcoreMesh(core_axis_name="c", subcore_axis_name="s")

    @functools.partial(
        pl.kernel,
        mesh=mesh,
        compiler_params=pltpu.CompilerParams(use_tc_tiling_on_sc=False,
                                             needs_layout_passes=False),
        out_type=(
            jax.ShapeDtypeStruct((N_NODES, DH), jnp.float32),
            jax.ShapeDtypeStruct((N_NODES, DH), jnp.float32),
            jax.ShapeDtypeStruct((CROWS, DH), jnp.float32),
        ),
        scratch_types=[
            pltpu.VMEM((G, K), jnp.int32),
            pltpu.VMEM((G, K), jnp.int32),
            pltpu.VMEM((K, DH), jnp.float32),
            pltpu.VMEM((CG,), jnp.int32),
            pltpu.VMEM((CROWS, DH), jnp.float32),
            pltpu.VMEM((CROWS,), jnp.int32),
            pltpu.VMEM_SHARED((N_NODES, DH), jnp.float32),
            pltpu.VMEM_SHARED((CROWS, DH), jnp.float32),
            pltpu.SemaphoreType.DMA,
        ],
    )
    def k(x0_hbm, x1_hbm, src_hbm, dst_hbm, dst1_hbm, out0, out1, cnt_out,
          sidx, didx, rows, dchunk, blk, riota, acc, cnt_sp, sem):
        c = lax.axis_index("c")
        s = lax.axis_index("s")

        zero16 = jnp.zeros((16,), jnp.float32)
        one16 = jnp.ones((16,), jnp.float32)

        def zrow(i, carry):
            for j in range(DH // 16):
                rows[i, pl.ds(j * 16, 16)] = zero16
            return carry

        lax.fori_loop(0, K, zrow, 0)

        def brow(i, carry):
            for j in range(DH // 16):
                blk[i, pl.ds(j * 16, 16)] = zero16
            return carry

        lax.fori_loop(0, CROWS, brow, 0)

        for j in range(CROWS // 16):
            riota[pl.ds(j * 16, 16)] = lax.iota(jnp.int32, 16) + j * 16

        for j in range(NPT // K):
            pltpu.sync_copy(rows, acc.at[pl.ds(s * NPT + j * K, K)])

        @pl.when(s == 0)
        def _():
            pltpu.sync_copy(rows.at[pl.ds(0, CROWS)], cnt_sp)

        plsc.subcore_barrier()

        def cgroup(g, carry):
            pltpu.sync_copy(dst1_hbm.at[pl.ds(s * EPT + g * CG, CG)], dchunk)

            def cbody(j, carry2):
                idx = dchunk[pl.ds(j * 16, 16)]
                plsc.addupdate_scatter(
                    blk,
                    [lax.shift_right_logical(idx, 7),
                     lax.bitwise_and(idx, 127)],
                    one16,
                )
                return carry2

            lax.fori_loop(0, CG // 16, cbody, 0)
            return carry

        lax.fori_loop(0, EPT // CG, cgroup, 0)

        def edge_loop(x_ref):
            def group(g, carry):
                pltpu.sync_copy(src_hbm.at[pl.ds(s * TROWS + g * G, G)], sidx)
                pltpu.sync_copy(dst_hbm.at[pl.ds(s * TROWS + g * G, G)], didx)

                def body(j, carry2):
                    pltpu.async_copy(x_ref.at[sidx.at[j]], rows, sem).wait()
                    pltpu.sync_copy(rows, acc.at[didx.at[j]], add=True)
                    return carry2

                lax.fori_loop(0, G, body, 0)
                return carry

            lax.fori_loop(0, TROWS // G, group, 0)

        @pl.when(c == 0)
        def _():
            edge_loop(x0_hbm)

        @pl.when(c == 1)
        def _():
            edge_loop(x1_hbm)

        plsc.subcore_barrier()

        pltpu.sync_copy(blk, cnt_sp.at[riota], add=True)
        plsc.subcore_barrier()

        def copy_out(dst_hbm_ref):
            for j in range(NPT // K):
                pltpu.sync_copy(acc.at[pl.ds(s * NPT + j * K, K)], rows)
                pltpu.sync_copy(rows, dst_hbm_ref.at[pl.ds(s * NPT + j * K, K)])

        @pl.when(c == 0)
        def _():
            copy_out(out0)

        @pl.when(c == 1)
        def _():
            copy_out(out1)

        @pl.when(jnp.logical_and(c == 0, s == 0))
        def _():
            pltpu.sync_copy(cnt_sp, blk)
            pltpu.sync_copy(blk, cnt_out)

    return k(x0, x1, src2, dst2, dst1)


BM = 400


def _tc_body(x_ref, nb0_ref, nb1_ref, cnt_ref, wst_ref, wnt0_ref, wnt1_ref,
             b_ref, g_ref, be_ref, o_ref):
    hs = jnp.dot(x_ref[...], wst_ref[...], preferred_element_type=jnp.float32)
    hn = (jnp.dot(nb0_ref[...], wnt0_ref[...], preferred_element_type=jnp.float32)
          + jnp.dot(nb1_ref[...], wnt1_ref[...], preferred_element_type=jnp.float32))
    inv = 1.0 / jnp.maximum(cnt_ref[...], 1.0)
    h = hs + hn * inv + b_ref[...]
    mu = jnp.mean(h, axis=-1, keepdims=True)
    d = h - mu
    var = jnp.mean(d * d, axis=-1, keepdims=True)
    o_ref[...] = d * lax.rsqrt(var + 1e-5) * g_ref[...] + be_ref[...]


def _tc_combine(x, nb0, nb1, cnt, wst, wnt0, wnt1, bias, gamma, beta):
    grid = (N_NODES // BM,)
    return pl.pallas_call(
        _tc_body,
        grid=grid,
        in_specs=[
            pl.BlockSpec((BM, D), lambda i: (i, 0)),
            pl.BlockSpec((BM, DH), lambda i: (i, 0)),
            pl.BlockSpec((BM, DH), lambda i: (i, 0)),
            pl.BlockSpec((BM, 1), lambda i: (i, 0)),
            pl.BlockSpec((D, D), lambda i: (0, 0)),
            pl.BlockSpec((DH, D), lambda i: (0, 0)),
            pl.BlockSpec((DH, D), lambda i: (0, 0)),
            pl.BlockSpec((1, D), lambda i: (0, 0)),
            pl.BlockSpec((1, D), lambda i: (0, 0)),
            pl.BlockSpec((1, D), lambda i: (0, 0)),
        ],
        out_specs=pl.BlockSpec((BM, D), lambda i: (i, 0)),
        out_shape=jax.ShapeDtypeStruct((N_NODES, D), jnp.float32),
    )(x, nb0, nb1, cnt, wst, wnt0, wnt1, bias, gamma, beta)


@jax.jit
def kernel(x, edge_index, deg, W_self, W_neigh, bias, ln_gamma, ln_beta):
    del deg
    x0 = x[:, :DH]
    x1 = x[:, DH:]
    src2 = edge_index[1].reshape(ROWS, K)
    dst2 = edge_index[0].reshape(ROWS, K)
    dst1 = edge_index[0]
    nb0, nb1, cnt_tab = _sc_segment_sum(x0, x1, src2, dst2, dst1)
    cnt = cnt_tab.reshape(NPAD)[:N_NODES, None]
    wst = W_self.T
    wnt = W_neigh.T
    return _tc_combine(x, nb0, nb1, cnt, wst, wnt[:DH], wnt[DH:],
                       bias[None, :], ln_gamma[None, :], ln_beta[None, :])


# speedup vs baseline: 7.0642x; 1.2686x over previous
import functools

import jax
import jax.numpy as jnp
from jax import lax
from jax.experimental import pallas as pl
from jax.experimental.pallas import tpu as pltpu
from jax.experimental.pallas import tpu_sc as plsc

N_NODES = 10000
NPAD = 10240
D = 256
DH = 128
E = 160000
K = 125
ROWS = E // K
NS = 16
TROWS = ROWS // NS
EPT = E // NS
NPT = N_NODES // NS
G = 8
CG = 2000
CROWS = NPAD // DH


def _sc_segment_sum(x0, x1, src2, dst2, dst1):
    mesh = plsc.VectorSubcoreMesh(core_axis_name="c", subcore_axis_name="s")

    @functools.partial(
        pl.kernel,
        mesh=mesh,
        compiler_params=pltpu.CompilerParams(use_tc_tiling_on_sc=False,
                                             needs_layout_passes=False),
        out_type=(
            jax.ShapeDtypeStruct((N_NODES, DH), jnp.float32),
            jax.ShapeDtypeStruct((N_NODES, DH), jnp.float32),
            jax.ShapeDtypeStruct((CROWS, DH), jnp.float32),
        ),
        scratch_types=[
            pltpu.VMEM((G, K), jnp.int32),
            pltpu.VMEM((G, K), jnp.int32),
            pltpu.VMEM((K, DH), jnp.float32),
            pltpu.VMEM((K, DH), jnp.float32),
            pltpu.VMEM((CG,), jnp.int32),
            pltpu.VMEM((CROWS, DH), jnp.float32),
            pltpu.VMEM((CROWS,), jnp.int32),
            pltpu.VMEM_SHARED((N_NODES, DH), jnp.float32),
            pltpu.VMEM_SHARED((CROWS, DH), jnp.float32),
            pltpu.SemaphoreType.DMA,
            pltpu.SemaphoreType.DMA,
        ],
    )
    def k(x0_hbm, x1_hbm, src_hbm, dst_hbm, dst1_hbm, out0, out1, cnt_out,
          sidx, didx, rows, rows2, dchunk, blk, riota, acc, cnt_sp, sem, sem2):
        c = lax.axis_index("c")
        s = lax.axis_index("s")

        zero16 = jnp.zeros((16,), jnp.float32)
        one16 = jnp.ones((16,), jnp.float32)

        def zrow(i, carry):
            for j in range(DH // 16):
                rows[i, pl.ds(j * 16, 16)] = zero16
            return carry

        lax.fori_loop(0, K, zrow, 0)

        def brow(i, carry):
            for j in range(DH // 16):
                blk[i, pl.ds(j * 16, 16)] = zero16
            return carry

        lax.fori_loop(0, CROWS, brow, 0)

        for j in range(CROWS // 16):
            riota[pl.ds(j * 16, 16)] = lax.iota(jnp.int32, 16) + j * 16

        for j in range(NPT // K):
            pltpu.sync_copy(rows, acc.at[pl.ds(s * NPT + j * K, K)])

        @pl.when(s == 0)
        def _():
            pltpu.sync_copy(rows.at[pl.ds(0, CROWS)], cnt_sp)

        plsc.subcore_barrier()

        def cgroup(g, carry):
            pltpu.sync_copy(dst1_hbm.at[pl.ds(s * EPT + g * CG, CG)], dchunk)

            def cbody(j, carry2):
                idx = dchunk[pl.ds(j * 16, 16)]
                plsc.addupdate_scatter(
                    blk,
                    [lax.shift_right_logical(idx, 7),
                     lax.bitwise_and(idx, 127)],
                    one16,
                )
                return carry2

            lax.fori_loop(0, CG // 16, cbody, 0)
            return carry

        lax.fori_loop(0, EPT // CG, cgroup, 0)

        def edge_loop(x_ref):
            def group(g, carry):
                pltpu.sync_copy(src_hbm.at[pl.ds(s * TROWS + g * G, G)], sidx)
                pltpu.sync_copy(dst_hbm.at[pl.ds(s * TROWS + g * G, G)], didx)
                pltpu.async_copy(x_ref.at[sidx.at[0]], rows, sem)

                def pair(p, carry2):
                    pltpu.async_copy(x_ref.at[sidx.at[2 * p + 1]], rows2, sem2)
                    pltpu.make_async_copy(x_ref.at[sidx.at[2 * p]], rows,
                                          sem).wait()
                    pltpu.sync_copy(rows, acc.at[didx.at[2 * p]], add=True)

                    @pl.when(p < G // 2 - 1)
                    def _():
                        pltpu.async_copy(x_ref.at[sidx.at[2 * p + 2]], rows,
                                         sem)

                    pltpu.make_async_copy(x_ref.at[sidx.at[2 * p + 1]], rows2,
                                          sem2).wait()
                    pltpu.sync_copy(rows2, acc.at[didx.at[2 * p + 1]],
                                    add=True)
                    return carry2

                lax.fori_loop(0, G // 2, pair, 0)
                return carry

            lax.fori_loop(0, TROWS // G, group, 0)

        @pl.when(c == 0)
        def _():
            edge_loop(x0_hbm)

        @pl.when(c == 1)
        def _():
            edge_loop(x1_hbm)

        plsc.subcore_barrier()

        pltpu.sync_copy(blk, cnt_sp.at[riota], add=True)
        plsc.subcore_barrier()

        def copy_out(dst_hbm_ref):
            for j in range(NPT // K):
                pltpu.sync_copy(acc.at[pl.ds(s * NPT + j * K, K)], rows)
                pltpu.sync_copy(rows, dst_hbm_ref.at[pl.ds(s * NPT + j * K, K)])

        @pl.when(c == 0)
        def _():
            copy_out(out0)

        @pl.when(c == 1)
        def _():
            copy_out(out1)

        @pl.when(jnp.logical_and(c == 0, s == 0))
        def _():
            pltpu.sync_copy(cnt_sp, blk)
            pltpu.sync_copy(blk, cnt_out)

    return k(x0, x1, src2, dst2, dst1)


BM = 400


def _tc_hself_body(x_ref, wst_ref, o_ref):
    o_ref[...] = jnp.dot(x_ref[...], wst_ref[...],
                         preferred_element_type=jnp.float32)


def _tc_hself(x, wst):
    return pl.pallas_call(
        _tc_hself_body,
        grid=(N_NODES // BM,),
        in_specs=[
            pl.BlockSpec((BM, D), lambda i: (i, 0)),
            pl.BlockSpec((D, D), lambda i: (0, 0)),
        ],
        out_specs=pl.BlockSpec((BM, D), lambda i: (i, 0)),
        out_shape=jax.ShapeDtypeStruct((N_NODES, D), jnp.float32),
    )(x, wst)


def _tc_body(hs_ref, nb0_ref, nb1_ref, cnt_ref, wnt0_ref, wnt1_ref,
             b_ref, g_ref, be_ref, o_ref):
    hn = (jnp.dot(nb0_ref[...], wnt0_ref[...], preferred_element_type=jnp.float32)
          + jnp.dot(nb1_ref[...], wnt1_ref[...], preferred_element_type=jnp.float32))
    inv = 1.0 / jnp.maximum(cnt_ref[...], 1.0)
    h = hs_ref[...] + hn * inv + b_ref[...]
    mu = jnp.mean(h, axis=-1, keepdims=True)
    d = h - mu
    var = jnp.mean(d * d, axis=-1, keepdims=True)
    o_ref[...] = d * lax.rsqrt(var + 1e-5) * g_ref[...] + be_ref[...]


def _tc_combine(hs, nb0, nb1, cnt, wnt0, wnt1, bias, gamma, beta):
    grid = (N_NODES // BM,)
    return pl.pallas_call(
        _tc_body,
        grid=grid,
        in_specs=[
            pl.BlockSpec((BM, D), lambda i: (i, 0)),
            pl.BlockSpec((BM, DH), lambda i: (i, 0)),
            pl.BlockSpec((BM, DH), lambda i: (i, 0)),
            pl.BlockSpec((BM, 1), lambda i: (i, 0)),
            pl.BlockSpec((DH, D), lambda i: (0, 0)),
            pl.BlockSpec((DH, D), lambda i: (0, 0)),
            pl.BlockSpec((1, D), lambda i: (0, 0)),
            pl.BlockSpec((1, D), lambda i: (0, 0)),
            pl.BlockSpec((1, D), lambda i: (0, 0)),
        ],
        out_specs=pl.BlockSpec((BM, D), lambda i: (i, 0)),
        out_shape=jax.ShapeDtypeStruct((N_NODES, D), jnp.float32),
    )(hs, nb0, nb1, cnt, wnt0, wnt1, bias, gamma, beta)


@jax.jit
def kernel(x, edge_index, deg, W_self, W_neigh, bias, ln_gamma, ln_beta):
    del deg
    x0 = x[:, :DH]
    x1 = x[:, DH:]
    src2 = edge_index[1].reshape(ROWS, K)
    dst2 = edge_index[0].reshape(ROWS, K)
    dst1 = edge_index[0]
    nb0, nb1, cnt_tab = _sc_segment_sum(x0, x1, src2, dst2, dst1)
    hs = _tc_hself(x, W_self.T)
    cnt = cnt_tab.reshape(NPAD)[:N_NODES, None]
    wnt = W_neigh.T
    return _tc_combine(hs, nb0, nb1, cnt, wnt[:DH], wnt[DH:],
                       bias[None, :], ln_gamma[None, :], ln_beta[None, :])


# speedup vs baseline: 7.4751x; 1.0582x over previous
import functools

import jax
import jax.numpy as jnp
from jax import lax
from jax.experimental import pallas as pl
from jax.experimental.pallas import tpu as pltpu
from jax.experimental.pallas import tpu_sc as plsc

N_NODES = 10000
NPAD = 10240
D = 256
DH = 128
E = 160000
K = 125
ROWS = E // K
NS = 16
TROWS = ROWS // NS
EPT = E // NS
NPT = N_NODES // NS
G = 16
CG = 2000
CROWS = NPAD // DH


def _sc_segment_sum(x0, x1, src2, dst2, dst1):
    mesh = plsc.VectorSubcoreMesh(core_axis_name="c", subcore_axis_name="s")

    @functools.partial(
        pl.kernel,
        mesh=mesh,
        compiler_params=pltpu.CompilerParams(use_tc_tiling_on_sc=False,
                                             needs_layout_passes=False),
        out_type=(
            jax.ShapeDtypeStruct((N_NODES, DH), jnp.float32),
            jax.ShapeDtypeStruct((N_NODES, DH), jnp.float32),
            jax.ShapeDtypeStruct((CROWS, DH), jnp.float32),
        ),
        scratch_types=[
            pltpu.VMEM((G, K), jnp.int32),
            pltpu.VMEM((G, K), jnp.int32),
            pltpu.VMEM((K, DH), jnp.float32),
            pltpu.VMEM((K, DH), jnp.float32),
            pltpu.VMEM((CG,), jnp.int32),
            pltpu.VMEM((CROWS, DH), jnp.float32),
            pltpu.VMEM((CROWS,), jnp.int32),
            pltpu.VMEM_SHARED((N_NODES, DH), jnp.float32),
            pltpu.VMEM_SHARED((CROWS, DH), jnp.float32),
            pltpu.SemaphoreType.DMA,
            pltpu.SemaphoreType.DMA,
        ],
    )
    def k(x0_hbm, x1_hbm, src_hbm, dst_hbm, dst1_hbm, out0, out1, cnt_out,
          sidx, didx, rows, rows2, dchunk, blk, riota, acc, cnt_sp, sem, sem2):
        c = lax.axis_index("c")
        s = lax.axis_index("s")

        zero16 = jnp.zeros((16,), jnp.float32)
        one16 = jnp.ones((16,), jnp.float32)

        def zrow(i, carry):
            for j in range(DH // 16):
                rows[i, pl.ds(j * 16, 16)] = zero16
            return carry

        lax.fori_loop(0, K, zrow, 0)

        def brow(i, carry):
            for j in range(DH // 16):
                blk[i, pl.ds(j * 16, 16)] = zero16
            return carry

        lax.fori_loop(0, CROWS, brow, 0)

        for j in range(CROWS // 16):
            riota[pl.ds(j * 16, 16)] = lax.iota(jnp.int32, 16) + j * 16

        for j in range(NPT // K):
            pltpu.sync_copy(rows, acc.at[pl.ds(s * NPT + j * K, K)])

        @pl.when(s == 0)
        def _():
            pltpu.sync_copy(rows.at[pl.ds(0, CROWS)], cnt_sp)

        plsc.subcore_barrier()

        def cgroup(g, carry):
            pltpu.sync_copy(dst1_hbm.at[pl.ds(s * EPT + g * CG, CG)], dchunk)

            def cbody(j, carry2):
                idx = dchunk[pl.ds(j * 16, 16)]
                plsc.addupdate_scatter(
                    blk,
                    [lax.shift_right_logical(idx, 7),
                     lax.bitwise_and(idx, 127)],
                    one16,
                )
                return carry2

            lax.fori_loop(0, CG // 16, cbody, 0)
            return carry

        lax.fori_loop(0, EPT // CG, cgroup, 0)

        def edge_loop(x_ref):
            def group(g, carry):
                pltpu.sync_copy(src_hbm.at[pl.ds(s * TROWS + g * G, G)], sidx)
                pltpu.sync_copy(dst_hbm.at[pl.ds(s * TROWS + g * G, G)], didx)
                pltpu.async_copy(x_ref.at[sidx.at[0]], rows, sem)

                def pair(p, carry2):
                    pltpu.async_copy(x_ref.at[sidx.at[2 * p + 1]], rows2, sem2)
                    pltpu.make_async_copy(x_ref.at[sidx.at[2 * p]], rows,
                                          sem).wait()
                    pltpu.sync_copy(rows, acc.at[didx.at[2 * p]], add=True)

                    @pl.when(p < G // 2 - 1)
                    def _():
                        pltpu.async_copy(x_ref.at[sidx.at[2 * p + 2]], rows,
                                         sem)

                    pltpu.make_async_copy(x_ref.at[sidx.at[2 * p + 1]], rows2,
                                          sem2).wait()
                    pltpu.sync_copy(rows2, acc.at[didx.at[2 * p + 1]],
                                    add=True)
                    return carry2

                lax.fori_loop(0, G // 2, pair, 0)
                return carry

            lax.fori_loop(0, TROWS // G, group, 0)

        @pl.when(c == 0)
        def _():
            edge_loop(x0_hbm)

        @pl.when(c == 1)
        def _():
            edge_loop(x1_hbm)

        plsc.subcore_barrier()

        pltpu.sync_copy(blk, cnt_sp.at[riota], add=True)
        plsc.subcore_barrier()

        def copy_out(dst_hbm_ref):
            for j in range(NPT // K):
                pltpu.sync_copy(acc.at[pl.ds(s * NPT + j * K, K)], rows)
                pltpu.sync_copy(rows, dst_hbm_ref.at[pl.ds(s * NPT + j * K, K)])

        @pl.when(c == 0)
        def _():
            copy_out(out0)

        @pl.when(c == 1)
        def _():
            copy_out(out1)

        @pl.when(jnp.logical_and(c == 0, s == 0))
        def _():
            pltpu.sync_copy(cnt_sp, blk)
            pltpu.sync_copy(blk, cnt_out)

    return k(x0, x1, src2, dst2, dst1)


BM = 400


def _tc_hself_body(x_ref, wst_ref, o_ref):
    o_ref[...] = jnp.dot(x_ref[...], wst_ref[...],
                         preferred_element_type=jnp.float32)


def _tc_hself(x, wst):
    return pl.pallas_call(
        _tc_hself_body,
        grid=(N_NODES // BM,),
        in_specs=[
            pl.BlockSpec((BM, D), lambda i: (i, 0)),
            pl.BlockSpec((D, D), lambda i: (0, 0)),
        ],
        out_specs=pl.BlockSpec((BM, D), lambda i: (i, 0)),
        out_shape=jax.ShapeDtypeStruct((N_NODES, D), jnp.float32),
    )(x, wst)


def _tc_body(hs_ref, nb0_ref, nb1_ref, cnt_ref, wnt0_ref, wnt1_ref,
             b_ref, g_ref, be_ref, o_ref):
    hn = (jnp.dot(nb0_ref[...], wnt0_ref[...], preferred_element_type=jnp.float32)
          + jnp.dot(nb1_ref[...], wnt1_ref[...], preferred_element_type=jnp.float32))
    inv = 1.0 / jnp.maximum(cnt_ref[...], 1.0)
    h = hs_ref[...] + hn * inv + b_ref[...]
    mu = jnp.mean(h, axis=-1, keepdims=True)
    d = h - mu
    var = jnp.mean(d * d, axis=-1, keepdims=True)
    o_ref[...] = d * lax.rsqrt(var + 1e-5) * g_ref[...] + be_ref[...]


def _tc_combine(hs, nb0, nb1, cnt, wnt0, wnt1, bias, gamma, beta):
    grid = (N_NODES // BM,)
    return pl.pallas_call(
        _tc_body,
        grid=grid,
        in_specs=[
            pl.BlockSpec((BM, D), lambda i: (i, 0)),
            pl.BlockSpec((BM, DH), lambda i: (i, 0)),
            pl.BlockSpec((BM, DH), lambda i: (i, 0)),
            pl.BlockSpec((BM, 1), lambda i: (i, 0)),
            pl.BlockSpec((DH, D), lambda i: (0, 0)),
            pl.BlockSpec((DH, D), lambda i: (0, 0)),
            pl.BlockSpec((1, D), lambda i: (0, 0)),
            pl.BlockSpec((1, D), lambda i: (0, 0)),
            pl.BlockSpec((1, D), lambda i: (0, 0)),
        ],
        out_specs=pl.BlockSpec((BM, D), lambda i: (i, 0)),
        out_shape=jax.ShapeDtypeStruct((N_NODES, D), jnp.float32),
    )(hs, nb0, nb1, cnt, wnt0, wnt1, bias, gamma, beta)


@jax.jit
def kernel(x, edge_index, deg, W_self, W_neigh, bias, ln_gamma, ln_beta):
    del deg
    x0 = x[:, :DH]
    x1 = x[:, DH:]
    src2 = edge_index[1].reshape(ROWS, K)
    dst2 = edge_index[0].reshape(ROWS, K)
    dst1 = edge_index[0]
    nb0, nb1, cnt_tab = _sc_segment_sum(x0, x1, src2, dst2, dst1)
    hs = _tc_hself(x, W_self.T)
    cnt = cnt_tab.reshape(NPAD)[:N_NODES, None]
    wnt = W_neigh.T
    return _tc_combine(hs, nb0, nb1, cnt, wnt[:DH], wnt[DH:],
                       bias[None, :], ln_gamma[None, :], ln_beta[None, :])


# speedup vs baseline: 7.8366x; 1.0484x over previous
import functools

import jax
import jax.numpy as jnp
from jax import lax
from jax.experimental import pallas as pl
from jax.experimental.pallas import tpu as pltpu
from jax.experimental.pallas import tpu_sc as plsc

N_NODES = 10000
NPAD = 10240
D = 256
DH = 128
E = 160000
K = 125
ROWS = E // K
NS = 16
TROWS = ROWS // NS
EPT = E // NS
NPT = N_NODES // NS
G = 16
CG = 2000
CROWS = NPAD // DH


def _sc_segment_sum(x0, x1, src2, dst2, dst1):
    mesh = plsc.VectorSubcoreMesh(core_axis_name="c", subcore_axis_name="s")

    @functools.partial(
        pl.kernel,
        mesh=mesh,
        compiler_params=pltpu.CompilerParams(use_tc_tiling_on_sc=False,
                                             needs_layout_passes=False),
        out_type=(
            jax.ShapeDtypeStruct((N_NODES, DH), jnp.float32),
            jax.ShapeDtypeStruct((N_NODES, DH), jnp.float32),
            jax.ShapeDtypeStruct((CROWS, DH), jnp.float32),
        ),
        scratch_types=[
            pltpu.VMEM((G, K), jnp.int32),
            pltpu.VMEM((G, K), jnp.int32),
            pltpu.VMEM((K, DH), jnp.float32),
            pltpu.VMEM((K, DH), jnp.float32),
            pltpu.VMEM((CG,), jnp.int32),
            pltpu.VMEM((CROWS, DH), jnp.float32),
            pltpu.VMEM((CROWS,), jnp.int32),
            pltpu.VMEM_SHARED((N_NODES, DH), jnp.float32),
            pltpu.VMEM_SHARED((CROWS, DH), jnp.float32),
            pltpu.SemaphoreType.DMA,
            pltpu.SemaphoreType.DMA,
        ],
    )
    def k(x0_hbm, x1_hbm, src_hbm, dst_hbm, dst1_hbm, out0, out1, cnt_out,
          sidx, didx, rows, rows2, dchunk, blk, riota, acc, cnt_sp, sem, sem2):
        c = lax.axis_index("c")
        s = lax.axis_index("s")

        zero16 = jnp.zeros((16,), jnp.float32)
        one16 = jnp.ones((16,), jnp.float32)

        def zrow(i, carry):
            for j in range(DH // 16):
                rows[i, pl.ds(j * 16, 16)] = zero16
            return carry

        lax.fori_loop(0, K, zrow, 0)

        def brow(i, carry):
            for j in range(DH // 16):
                blk[i, pl.ds(j * 16, 16)] = zero16
            return carry

        lax.fori_loop(0, CROWS, brow, 0)

        for j in range(CROWS // 16):
            riota[pl.ds(j * 16, 16)] = lax.iota(jnp.int32, 16) + j * 16

        for j in range(NPT // K):
            pltpu.sync_copy(rows, acc.at[pl.ds(s * NPT + j * K, K)])

        @pl.when(s == 0)
        def _():
            pltpu.sync_copy(rows.at[pl.ds(0, CROWS)], cnt_sp)

        plsc.subcore_barrier()

        def edge_loop(x_ref):
            def group(g, carry):
                pltpu.sync_copy(src_hbm.at[pl.ds(s * TROWS + g * G, G)], sidx)
                pltpu.sync_copy(dst_hbm.at[pl.ds(s * TROWS + g * G, G)], didx)
                pltpu.sync_copy(dst1_hbm.at[pl.ds(s * EPT + g * CG, CG)],
                                dchunk)
                pltpu.async_copy(x_ref.at[sidx.at[0]], rows, sem)

                def cbody(j, carry2):
                    idx = dchunk[pl.ds(j * 16, 16)]
                    plsc.addupdate_scatter(
                        blk,
                        [lax.shift_right_logical(idx, 7),
                         lax.bitwise_and(idx, 127)],
                        one16,
                    )
                    return carry2

                lax.fori_loop(0, CG // 16, cbody, 0)

                def pair(p, carry2):
                    pltpu.async_copy(x_ref.at[sidx.at[2 * p + 1]], rows2, sem2)
                    pltpu.make_async_copy(x_ref.at[sidx.at[2 * p]], rows,
                                          sem).wait()
                    pltpu.sync_copy(rows, acc.at[didx.at[2 * p]], add=True)

                    @pl.when(p < G // 2 - 1)
                    def _():
                        pltpu.async_copy(x_ref.at[sidx.at[2 * p + 2]], rows,
                                         sem)

                    pltpu.make_async_copy(x_ref.at[sidx.at[2 * p + 1]], rows2,
                                          sem2).wait()
                    pltpu.sync_copy(rows2, acc.at[didx.at[2 * p + 1]],
                                    add=True)
                    return carry2

                lax.fori_loop(0, G // 2, pair, 0)
                return carry

            lax.fori_loop(0, TROWS // G, group, 0)

        @pl.when(c == 0)
        def _():
            edge_loop(x0_hbm)

        @pl.when(c == 1)
        def _():
            edge_loop(x1_hbm)

        plsc.subcore_barrier()

        pltpu.sync_copy(blk, cnt_sp.at[riota], add=True)
        plsc.subcore_barrier()

        def copy_out(dst_hbm_ref):
            for j in range(NPT // K):
                pltpu.sync_copy(acc.at[pl.ds(s * NPT + j * K, K)], rows)
                pltpu.sync_copy(rows, dst_hbm_ref.at[pl.ds(s * NPT + j * K, K)])

        @pl.when(c == 0)
        def _():
            copy_out(out0)

        @pl.when(c == 1)
        def _():
            copy_out(out1)

        @pl.when(jnp.logical_and(c == 0, s == 0))
        def _():
            pltpu.sync_copy(cnt_sp, blk)
            pltpu.sync_copy(blk, cnt_out)

    return k(x0, x1, src2, dst2, dst1)


BM = 400


def _tc_hself_body(x_ref, wst_ref, o_ref):
    o_ref[...] = jnp.dot(x_ref[...], wst_ref[...],
                         preferred_element_type=jnp.float32)


def _tc_hself(x, wst):
    return pl.pallas_call(
        _tc_hself_body,
        grid=(N_NODES // BM,),
        in_specs=[
            pl.BlockSpec((BM, D), lambda i: (i, 0)),
            pl.BlockSpec((D, D), lambda i: (0, 0)),
        ],
        out_specs=pl.BlockSpec((BM, D), lambda i: (i, 0)),
        out_shape=jax.ShapeDtypeStruct((N_NODES, D), jnp.float32),
    )(x, wst)


def _tc_body(hs_ref, nb0_ref, nb1_ref, cnt_ref, wnt0_ref, wnt1_ref,
             b_ref, g_ref, be_ref, o_ref):
    hn = (jnp.dot(nb0_ref[...], wnt0_ref[...], preferred_element_type=jnp.float32)
          + jnp.dot(nb1_ref[...], wnt1_ref[...], preferred_element_type=jnp.float32))
    inv = 1.0 / jnp.maximum(cnt_ref[...], 1.0)
    h = hs_ref[...] + hn * inv + b_ref[...]
    mu = jnp.mean(h, axis=-1, keepdims=True)
    d = h - mu
    var = jnp.mean(d * d, axis=-1, keepdims=True)
    o_ref[...] = d * lax.rsqrt(var + 1e-5) * g_ref[...] + be_ref[...]


def _tc_combine(hs, nb0, nb1, cnt, wnt0, wnt1, bias, gamma, beta):
    grid = (N_NODES // BM,)
    return pl.pallas_call(
        _tc_body,
        grid=grid,
        in_specs=[
            pl.BlockSpec((BM, D), lambda i: (i, 0)),
            pl.BlockSpec((BM, DH), lambda i: (i, 0)),
            pl.BlockSpec((BM, DH), lambda i: (i, 0)),
            pl.BlockSpec((BM, 1), lambda i: (i, 0)),
            pl.BlockSpec((DH, D), lambda i: (0, 0)),
            pl.BlockSpec((DH, D), lambda i: (0, 0)),
            pl.BlockSpec((1, D), lambda i: (0, 0)),
            pl.BlockSpec((1, D), lambda i: (0, 0)),
            pl.BlockSpec((1, D), lambda i: (0, 0)),
        ],
        out_specs=pl.BlockSpec((BM, D), lambda i: (i, 0)),
        out_shape=jax.ShapeDtypeStruct((N_NODES, D), jnp.float32),
    )(hs, nb0, nb1, cnt, wnt0, wnt1, bias, gamma, beta)


@jax.jit
def kernel(x, edge_index, deg, W_self, W_neigh, bias, ln_gamma, ln_beta):
    del deg
    x0 = x[:, :DH]
    x1 = x[:, DH:]
    src2 = edge_index[1].reshape(ROWS, K)
    dst2 = edge_index[0].reshape(ROWS, K)
    dst1 = edge_index[0]
    nb0, nb1, cnt_tab = _sc_segment_sum(x0, x1, src2, dst2, dst1)
    hs = _tc_hself(x, W_self.T)
    cnt = cnt_tab.reshape(NPAD)[:N_NODES, None]
    wnt = W_neigh.T
    return _tc_combine(hs, nb0, nb1, cnt, wnt[:DH], wnt[DH:],
                       bias[None, :], ln_gamma[None, :], ln_beta[None, :])


# speedup vs baseline: 7.8850x; 1.0062x over previous
import functools

import jax
import jax.numpy as jnp
from jax import lax
from jax.experimental import pallas as pl
from jax.experimental.pallas import tpu as pltpu
from jax.experimental.pallas import tpu_sc as plsc

N_NODES = 10000
NPAD = 10240
D = 256
DH = 128
E = 160000
K = 125
ROWS = E // K
NS = 16
TROWS = ROWS // NS
EPT = E // NS
NPT = N_NODES // NS
G = 16
CG = 2000
CROWS = NPAD // DH


def _sc_segment_sum(x0, x1, src2, dst2, dst1):
    mesh = plsc.VectorSubcoreMesh(core_axis_name="c", subcore_axis_name="s")

    @functools.partial(
        pl.kernel,
        mesh=mesh,
        compiler_params=pltpu.CompilerParams(use_tc_tiling_on_sc=False,
                                             needs_layout_passes=False),
        out_type=(
            jax.ShapeDtypeStruct((N_NODES, DH), jnp.float32),
            jax.ShapeDtypeStruct((N_NODES, DH), jnp.float32),
            jax.ShapeDtypeStruct((CROWS, DH), jnp.float32),
        ),
        scratch_types=[
            pltpu.VMEM((G, K), jnp.int32),
            pltpu.VMEM((G, K), jnp.int32),
            pltpu.VMEM((K, DH), jnp.float32),
            pltpu.VMEM((K, DH), jnp.float32),
            pltpu.VMEM((CG,), jnp.int32),
            pltpu.VMEM((CROWS, DH), jnp.float32),
            pltpu.VMEM((CROWS,), jnp.int32),
            pltpu.VMEM_SHARED((N_NODES, DH), jnp.float32),
            pltpu.VMEM_SHARED((CROWS, DH), jnp.float32),
            pltpu.SemaphoreType.DMA,
            pltpu.SemaphoreType.DMA,
        ],
    )
    def k(x0_hbm, x1_hbm, src_hbm, dst_hbm, dst1_hbm, out0, out1, cnt_out,
          sidx, didx, rows, rows2, dchunk, blk, riota, acc, cnt_sp, sem, sem2):
        c = lax.axis_index("c")
        s = lax.axis_index("s")

        zero16 = jnp.zeros((16,), jnp.float32)
        one16 = jnp.ones((16,), jnp.float32)

        def zrow(i, carry):
            for j in range(DH // 16):
                rows[i, pl.ds(j * 16, 16)] = zero16
            return carry

        lax.fori_loop(0, K, zrow, 0)

        def brow(i, carry):
            for j in range(DH // 16):
                blk[i, pl.ds(j * 16, 16)] = zero16
            return carry

        lax.fori_loop(0, CROWS, brow, 0)

        for j in range(CROWS // 16):
            riota[pl.ds(j * 16, 16)] = lax.iota(jnp.int32, 16) + j * 16

        for j in range(NPT // K):
            pltpu.sync_copy(rows, acc.at[pl.ds(s * NPT + j * K, K)])

        @pl.when(s == 0)
        def _():
            pltpu.sync_copy(rows.at[pl.ds(0, CROWS)], cnt_sp)

        plsc.subcore_barrier()

        def edge_loop(x_ref):
            def group(g, carry):
                pltpu.sync_copy(src_hbm.at[pl.ds(s * TROWS + g * G, G)], sidx)
                pltpu.sync_copy(dst_hbm.at[pl.ds(s * TROWS + g * G, G)], didx)
                pltpu.sync_copy(dst1_hbm.at[pl.ds(s * EPT + g * CG, CG)],
                                dchunk)
                pltpu.async_copy(x_ref.at[sidx.at[0]], rows, sem)

                def cbody(j, carry2):
                    idx = dchunk[pl.ds(j * 16, 16)]
                    plsc.addupdate_scatter(
                        blk,
                        [lax.shift_right_logical(idx, 7),
                         lax.bitwise_and(idx, 127)],
                        one16,
                    )
                    return carry2

                lax.fori_loop(0, CG // 16, cbody, 0)

                def pair(p, carry2):
                    pltpu.async_copy(x_ref.at[sidx.at[2 * p + 1]], rows2, sem2)
                    pltpu.make_async_copy(x_ref.at[sidx.at[2 * p]], rows,
                                          sem).wait()
                    pltpu.sync_copy(rows, acc.at[didx.at[2 * p]], add=True)

                    @pl.when(p < G // 2 - 1)
                    def _():
                        pltpu.async_copy(x_ref.at[sidx.at[2 * p + 2]], rows,
                                         sem)

                    pltpu.make_async_copy(x_ref.at[sidx.at[2 * p + 1]], rows2,
                                          sem2).wait()
                    pltpu.sync_copy(rows2, acc.at[didx.at[2 * p + 1]],
                                    add=True)
                    return carry2

                lax.fori_loop(0, G // 2, pair, 0)
                return carry

            lax.fori_loop(0, TROWS // G, group, 0)

        @pl.when(c == 0)
        def _():
            edge_loop(x0_hbm)

        @pl.when(c == 1)
        def _():
            edge_loop(x1_hbm)

        plsc.subcore_barrier()

        pltpu.sync_copy(blk, cnt_sp.at[riota], add=True)
        plsc.subcore_barrier()

        def copy_out(dst_hbm_ref):
            for j in range(NPT // K):
                pltpu.sync_copy(acc.at[pl.ds(s * NPT + j * K, K)], rows)
                pltpu.sync_copy(rows, dst_hbm_ref.at[pl.ds(s * NPT + j * K, K)])

        @pl.when(c == 0)
        def _():
            copy_out(out0)

        @pl.when(c == 1)
        def _():
            copy_out(out1)

        @pl.when(jnp.logical_and(c == 0, s == 0))
        def _():
            pltpu.sync_copy(cnt_sp, blk)
            pltpu.sync_copy(blk, cnt_out)

    return k(x0, x1, src2, dst2, dst1)


BM = 400


def _tc_body(x_ref, nb0_ref, nb1_ref, cnt_ref, wst_ref, wnt0_ref, wnt1_ref,
             b_ref, g_ref, be_ref, o_ref):
    hs = jnp.dot(x_ref[...], wst_ref[...], preferred_element_type=jnp.float32)
    hn = (jnp.dot(nb0_ref[...], wnt0_ref[...], preferred_element_type=jnp.float32)
          + jnp.dot(nb1_ref[...], wnt1_ref[...], preferred_element_type=jnp.float32))
    inv = 1.0 / jnp.maximum(cnt_ref[...], 1.0)
    h = hs + hn * inv + b_ref[...]
    mu = jnp.mean(h, axis=-1, keepdims=True)
    d = h - mu
    var = jnp.mean(d * d, axis=-1, keepdims=True)
    o_ref[...] = d * lax.rsqrt(var + 1e-5) * g_ref[...] + be_ref[...]


def _tc_combine(x, nb0, nb1, cnt, wst, wnt0, wnt1, bias, gamma, beta):
    grid = (N_NODES // BM,)
    return pl.pallas_call(
        _tc_body,
        grid=grid,
        in_specs=[
            pl.BlockSpec((BM, D), lambda i: (i, 0)),
            pl.BlockSpec((BM, DH), lambda i: (i, 0)),
            pl.BlockSpec((BM, DH), lambda i: (i, 0)),
            pl.BlockSpec((BM, 1), lambda i: (i, 0)),
            pl.BlockSpec((D, D), lambda i: (0, 0)),
            pl.BlockSpec((DH, D), lambda i: (0, 0)),
            pl.BlockSpec((DH, D), lambda i: (0, 0)),
            pl.BlockSpec((1, D), lambda i: (0, 0)),
            pl.BlockSpec((1, D), lambda i: (0, 0)),
            pl.BlockSpec((1, D), lambda i: (0, 0)),
        ],
        out_specs=pl.BlockSpec((BM, D), lambda i: (i, 0)),
        out_shape=jax.ShapeDtypeStruct((N_NODES, D), jnp.float32),
    )(x, nb0, nb1, cnt, wst, wnt0, wnt1, bias, gamma, beta)


@jax.jit
def kernel(x, edge_index, deg, W_self, W_neigh, bias, ln_gamma, ln_beta):
    del deg
    x0 = x[:, :DH]
    x1 = x[:, DH:]
    src2 = edge_index[1].reshape(ROWS, K)
    dst2 = edge_index[0].reshape(ROWS, K)
    dst1 = edge_index[0]
    nb0, nb1, cnt_tab = _sc_segment_sum(x0, x1, src2, dst2, dst1)
    cnt = cnt_tab.reshape(NPAD)[:N_NODES, None]
    wnt = W_neigh.T
    return _tc_combine(x, nb0, nb1, cnt, W_self.T, wnt[:DH], wnt[DH:],
                       bias[None, :], ln_gamma[None, :], ln_beta[None, :])
